# baseline (speedup 1.0000x reference)
import functools

import jax
import jax.numpy as jnp
from jax import lax
from jax.experimental import pallas as pl
from jax.experimental.pallas import tpu as pltpu, tpu_sc as plsc

N = 10000
NE = 320000
NC = 2
NS = 16
NW = NC * NS
CH = 128
NCHUNK = 80
NE_PAD = NW * NCHUNK * CH
N_PAD = 10240
ZR = N_PAD // NS
BR = 2000


def _make_sc_scatter(D, with_gather):
    mesh = plsc.VectorSubcoreMesh(core_axis_name="c", subcore_axis_name="s")
    scratch = [
        pltpu.VMEM((NCHUNK, CH), jnp.int32),
        pltpu.VMEM((NCHUNK, CH), jnp.int32),
        pltpu.VMEM((CH, D), jnp.float32),
        pltpu.VMEM((CH, D), jnp.float32),
        pltpu.VMEM_SHARED((N_PAD, D), jnp.float32),
        pltpu.VMEM_SHARED((N, D) if with_gather else (1, D), jnp.float32),
        pltpu.SemaphoreType.DMA,
        pltpu.SemaphoreType.DMA,
    ]

    @functools.partial(
        pl.kernel, mesh=mesh,
        out_type=jax.ShapeDtypeStruct((NC * N_PAD, D), jnp.float32),
        scratch_types=scratch,
        compiler_params=pltpu.CompilerParams(use_tc_tiling_on_sc=False),
    )
    def scat(src_hbm, dst_hbm, g_hbm, z_hbm, out_hbm, src_v, dst_v, rows0,
             rows1, acc, g_sh, sem0, sem1):
        cid = lax.axis_index("c")
        sid = lax.axis_index("s")
        wid = sid * NC + cid
        pltpu.sync_copy(dst_hbm.at[wid], dst_v)
        if with_gather:
            pltpu.sync_copy(src_hbm.at[wid], src_v)
            @pl.when(sid < 10)
            def _():
                pltpu.sync_copy(g_hbm.at[pl.ds(sid * 1000, 1000)],
                                g_sh.at[pl.ds(sid * 1000, 1000)])
        else:
            pltpu.sync_copy(g_hbm, rows0)
        pltpu.sync_copy(z_hbm.at[pl.ds(sid * ZR, ZR)],
                        acc.at[pl.ds(sid * ZR, ZR)])
        plsc.subcore_barrier()

        if with_gather:
            def body(i, carry):
                j = i * 2
                cp_b = pltpu.async_copy(g_hbm.at[src_v.at[j + 1]], rows1,
                                        sem1)
                cp_a = pltpu.async_copy(g_sh.at[src_v.at[j]], rows0, sem0)
                cp_a.wait()
                pltpu.sync_copy(rows0, acc.at[dst_v.at[j]], add=True)
                cp_b.wait()
                pltpu.sync_copy(rows1, acc.at[dst_v.at[j + 1]], add=True)
                return carry

            lax.fori_loop(0, NCHUNK // 2, body, 0)
        else:
            def body(j, carry):
                pltpu.sync_copy(rows0, acc.at[dst_v.at[j]], add=True)
                return carry

            lax.fori_loop(0, NCHUNK, body, 0)

        plsc.subcore_barrier()
        pltpu.sync_copy(acc.at[pl.ds(sid * ZR, ZR)],
                        out_hbm.at[pl.ds(cid * N_PAD + sid * ZR, ZR)])

    return scat


_sc_deg = _make_sc_scatter(16, with_gather=False)
_sc_scat64 = _make_sc_scatter(64, with_gather=True)
_sc_scat48 = _make_sc_scatter(48, with_gather=True)


def _full(shape):
    return pl.BlockSpec(shape, lambda i: (0,) * len(shape))


def _rows(d):
    return pl.BlockSpec((BR, d), lambda i: (i, 0))


def _inv_of(p0, p1):
    return lax.rsqrt(p0[:, :1] + p1[:, :1] + 1.0)


def _tc1_body(p0_r, p1_r, x_r, w1_r, b1_r, w2_r, b2_r, wg1_r, o_r):
    h = jnp.maximum(jnp.dot(x_r[...], w1_r[...],
                            preferred_element_type=jnp.float32) + b1_r[...], 0.0)
    h = jnp.maximum(jnp.dot(h, w2_r[...],
                            preferred_element_type=jnp.float32) + b2_r[...], 0.0)
    o_r[...] = jnp.dot(h, wg1_r[...],
                       preferred_element_type=jnp.float32) * _inv_of(
                           p0_r[...], p1_r[...])


def _tc3_body(p0_r, p1_r, s0_r, s1_r, g_r, b1_r, w2_r, o_r):
    inv = _inv_of(p0_r[...], p1_r[...])
    out1 = jnp.maximum((s0_r[...] + s1_r[...] + g_r[...]) * inv + b1_r[...], 0.0)
    o_r[...] = jnp.dot(out1, w2_r[...],
                       preferred_element_type=jnp.float32) * inv


def _tc4_body(p0_r, p1_r, s0_r, s1_r, g_r, b2_r, o_r):
    inv = _inv_of(p0_r[...], p1_r[...])
    z = ((s0_r[...] + s1_r[...] + g_r[...]) * inv)[:, :40] + b2_r[...]
    m = jnp.max(z, axis=1, keepdims=True)
    e = jnp.exp(z - m)
    o_r[...] = (z - m) - jnp.log(jnp.sum(e, axis=1, keepdims=True))


def kernel(x, edge_index, W_fse1, b_fse1, W_fse2, b_fse2, W_gc1, b_gc1,
           W_gc2, b_gc2):
    f32 = jnp.float32
    src = edge_index[0].astype(jnp.int32)
    dst = edge_index[1].astype(jnp.int32)
    src3 = jnp.concatenate([src, jnp.zeros((NE_PAD - NE,), jnp.int32)]
                           ).reshape(NW, NCHUNK, CH)
    dst3 = jnp.concatenate([dst, jnp.full((NE_PAD - NE,), N, jnp.int32)]
                           ).reshape(NW, NCHUNK, CH)

    grid = N // BR

    ones16 = jnp.ones((CH, 16), f32)
    z16 = jnp.zeros((N_PAD, 16), f32)
    degp = _sc_deg(src3, dst3, ones16, z16)
    p0, p1 = degp[:N], degp[N_PAD:N_PAD + N]

    g1 = pl.pallas_call(
        _tc1_body,
        grid=(grid,),
        in_specs=[_rows(16), _rows(16), _rows(128), _full((128, 16)),
                  _full((1, 16)), _full((16, 16)), _full((1, 16)),
                  _full((16, 64))],
        out_specs=_rows(64),
        out_shape=jax.ShapeDtypeStruct((N, 64), f32),
    )(p0, p1, x, W_fse1, b_fse1.reshape(1, 16), W_fse2,
      b_fse2.reshape(1, 16), W_gc1)

    z64 = jnp.zeros((N_PAD, 64), f32)
    s1 = _sc_scat64(src3, dst3, g1, z64)

    w2p = jnp.pad(W_gc2, ((0, 0), (0, 8)))
    g2 = pl.pallas_call(
        _tc3_body,
        grid=(grid,),
        in_specs=[_rows(16), _rows(16), _rows(64), _rows(64), _rows(64),
                  _full((1, 64)), _full((64, 48))],
        out_specs=_rows(48),
        out_shape=jax.ShapeDtypeStruct((N, 48), f32),
    )(p0, p1, s1[:N], s1[N_PAD:N_PAD + N], g1, b_gc1.reshape(1, 64), w2p)

    z48 = jnp.zeros((N_PAD, 48), f32)
    s2 = _sc_scat48(src3, dst3, g2, z48)

    out = pl.pallas_call(
        _tc4_body,
        grid=(grid,),
        in_specs=[_rows(16), _rows(16), _rows(48), _rows(48), _rows(48),
                  _full((1, 40))],
        out_specs=_rows(40),
        out_shape=jax.ShapeDtypeStruct((N, 40), f32),
    )(p0, p1, s2[:N], s2[N_PAD:N_PAD + N], g2, b_gc2.reshape(1, 40))

    return out


# speedup vs baseline: 1.3041x; 1.3041x over previous
import functools

import jax
import jax.numpy as jnp
from jax import lax
from jax.experimental import pallas as pl
from jax.experimental.pallas import tpu as pltpu, tpu_sc as plsc

N = 10000
NE = 320000
NC = 2
NS = 16
NW = NC * NS
CH = 128
NCHUNK = 79
NE_PAD = NW * NCHUNK * CH
N_PAD = 10240
ZR = N_PAD // NS
BR = 2000


def _make_sc_scatter(D, with_gather):
    mesh = plsc.VectorSubcoreMesh(core_axis_name="c", subcore_axis_name="s")
    scratch = [
        pltpu.VMEM((NCHUNK, CH), jnp.int32),
        pltpu.VMEM((NCHUNK, CH), jnp.int32),
        pltpu.VMEM((CH, D), jnp.float32),
        pltpu.VMEM((CH, D), jnp.float32),
        pltpu.VMEM_SHARED((N_PAD, D), jnp.float32),
        pltpu.VMEM_SHARED((N, D) if with_gather else (1, D), jnp.float32),
        pltpu.SemaphoreType.DMA,
        pltpu.SemaphoreType.DMA,
    ]

    @functools.partial(
        pl.kernel, mesh=mesh,
        out_type=jax.ShapeDtypeStruct((NC * N_PAD, D), jnp.float32),
        scratch_types=scratch,
        compiler_params=pltpu.CompilerParams(use_tc_tiling_on_sc=False),
    )
    def scat(src_hbm, dst_hbm, g_hbm, z_hbm, out_hbm, src_v, dst_v, rows0,
             rows1, acc, g_sh, sem0, sem1):
        cid = lax.axis_index("c")
        sid = lax.axis_index("s")
        wid = sid * NC + cid
        pltpu.sync_copy(dst_hbm.at[wid], dst_v)
        if with_gather:
            pltpu.sync_copy(src_hbm.at[wid], src_v)
            @pl.when(sid < 10)
            def _():
                pltpu.sync_copy(g_hbm.at[pl.ds(sid * 1000, 1000)],
                                g_sh.at[pl.ds(sid * 1000, 1000)])
        else:
            pltpu.sync_copy(g_hbm, rows0)
        pltpu.sync_copy(z_hbm.at[pl.ds(sid * ZR, ZR)],
                        acc.at[pl.ds(sid * ZR, ZR)])
        plsc.subcore_barrier()

        if with_gather:
            def body(j, carry):
                pltpu.async_copy(g_sh.at[src_v.at[j]], rows0, sem0).wait()
                pltpu.sync_copy(rows0, acc.at[dst_v.at[j]], add=True)
                return carry

            lax.fori_loop(0, NCHUNK, body, 0)
        else:
            def body(j, carry):
                pltpu.sync_copy(rows0, acc.at[dst_v.at[j]], add=True)
                return carry

            lax.fori_loop(0, NCHUNK, body, 0)

        plsc.subcore_barrier()
        pltpu.sync_copy(acc.at[pl.ds(sid * ZR, ZR)],
                        out_hbm.at[pl.ds(cid * N_PAD + sid * ZR, ZR)])

    return scat


_sc_deg = _make_sc_scatter(16, with_gather=False)
_sc_scat64 = _make_sc_scatter(64, with_gather=True)
_sc_scat48 = _make_sc_scatter(48, with_gather=True)


def _full(shape):
    return pl.BlockSpec(shape, lambda i: (0,) * len(shape))


def _rows(d):
    return pl.BlockSpec((BR, d), lambda i: (i, 0))


def _inv_of(p0, p1):
    return lax.rsqrt(p0[:, :1] + p1[:, :1] + 1.0)


def _tc1_body(p0_r, p1_r, x_r, w1_r, b1_r, w2_r, b2_r, wg1_r, o_r):
    h = jnp.maximum(jnp.dot(x_r[...], w1_r[...],
                            preferred_element_type=jnp.float32) + b1_r[...], 0.0)
    h = jnp.maximum(jnp.dot(h, w2_r[...],
                            preferred_element_type=jnp.float32) + b2_r[...], 0.0)
    o_r[...] = jnp.dot(h, wg1_r[...],
                       preferred_element_type=jnp.float32) * _inv_of(
                           p0_r[...], p1_r[...])


def _tc3_body(p0_r, p1_r, s0_r, s1_r, g_r, b1_r, w2_r, o_r):
    inv = _inv_of(p0_r[...], p1_r[...])
    out1 = jnp.maximum((s0_r[...] + s1_r[...] + g_r[...]) * inv + b1_r[...], 0.0)
    o_r[...] = jnp.dot(out1, w2_r[...],
                       preferred_element_type=jnp.float32) * inv


def _tc4_body(p0_r, p1_r, s0_r, s1_r, g_r, b2_r, o_r):
    inv = _inv_of(p0_r[...], p1_r[...])
    z = ((s0_r[...] + s1_r[...] + g_r[...]) * inv)[:, :40] + b2_r[...]
    m = jnp.max(z, axis=1, keepdims=True)
    e = jnp.exp(z - m)
    o_r[...] = (z - m) - jnp.log(jnp.sum(e, axis=1, keepdims=True))


def kernel(x, edge_index, W_fse1, b_fse1, W_fse2, b_fse2, W_gc1, b_gc1,
           W_gc2, b_gc2):
    f32 = jnp.float32
    src = edge_index[0].astype(jnp.int32)
    dst = edge_index[1].astype(jnp.int32)
    src3 = jnp.concatenate([src, jnp.zeros((NE_PAD - NE,), jnp.int32)]
                           ).reshape(NW, NCHUNK, CH)
    dst3 = jnp.concatenate([dst, jnp.full((NE_PAD - NE,), N, jnp.int32)]
                           ).reshape(NW, NCHUNK, CH)

    grid = N // BR

    ones16 = jnp.ones((CH, 16), f32)
    z16 = jnp.zeros((N_PAD, 16), f32)
    degp = _sc_deg(src3, dst3, ones16, z16)
    p0, p1 = degp[:N], degp[N_PAD:N_PAD + N]

    g1 = pl.pallas_call(
        _tc1_body,
        grid=(grid,),
        in_specs=[_rows(16), _rows(16), _rows(128), _full((128, 16)),
                  _full((1, 16)), _full((16, 16)), _full((1, 16)),
                  _full((16, 64))],
        out_specs=_rows(64),
        out_shape=jax.ShapeDtypeStruct((N, 64), f32),
    )(p0, p1, x, W_fse1, b_fse1.reshape(1, 16), W_fse2,
      b_fse2.reshape(1, 16), W_gc1)

    z64 = jnp.zeros((N_PAD, 64), f32)
    s1 = _sc_scat64(src3, dst3, g1, z64)

    w2p = jnp.pad(W_gc2, ((0, 0), (0, 8)))
    g2 = pl.pallas_call(
        _tc3_body,
        grid=(grid,),
        in_specs=[_rows(16), _rows(16), _rows(64), _rows(64), _rows(64),
                  _full((1, 64)), _full((64, 48))],
        out_specs=_rows(48),
        out_shape=jax.ShapeDtypeStruct((N, 48), f32),
    )(p0, p1, s1[:N], s1[N_PAD:N_PAD + N], g1, b_gc1.reshape(1, 64), w2p)

    z48 = jnp.zeros((N_PAD, 48), f32)
    s2 = _sc_scat48(src3, dst3, g2, z48)

    out = pl.pallas_call(
        _tc4_body,
        grid=(grid,),
        in_specs=[_rows(16), _rows(16), _rows(48), _rows(48), _rows(48),
                  _full((1, 40))],
        out_specs=_rows(40),
        out_shape=jax.ShapeDtypeStruct((N, 40), f32),
    )(p0, p1, s2[:N], s2[N_PAD:N_PAD + N], g2, b_gc2.reshape(1, 40))

    return out


# speedup vs baseline: 1.3465x; 1.0325x over previous
import functools

import jax
import jax.numpy as jnp
from jax import lax
from jax.experimental import pallas as pl
from jax.experimental.pallas import tpu as pltpu, tpu_sc as plsc

N = 10000
NE = 320000
NC = 2
NS = 16
NW = NC * NS
CH = 128
NCHUNK = 79
EPT = NE // NW
EPT_PAD = NCHUNK * CH
N_PAD = 10240
ZR = N_PAD // NS
BR = 2000


def _make_sc_scatter(D, with_gather):
    mesh = plsc.VectorSubcoreMesh(core_axis_name="c", subcore_axis_name="s")
    scratch = [
        pltpu.VMEM((EPT_PAD,), jnp.int32),
        pltpu.VMEM((EPT_PAD,), jnp.int32),
        pltpu.VMEM((CH, D), jnp.float32),
        pltpu.VMEM_SHARED((N_PAD, D), jnp.float32),
        pltpu.VMEM_SHARED((N, D) if with_gather else (1, D), jnp.float32),
        pltpu.SemaphoreType.DMA,
    ]

    @functools.partial(
        pl.kernel, mesh=mesh,
        out_type=jax.ShapeDtypeStruct((NC * N_PAD, D), jnp.float32),
        scratch_types=scratch,
        compiler_params=pltpu.CompilerParams(use_tc_tiling_on_sc=False),
    )
    def scat(edge_hbm, g_hbm, z_hbm, out_hbm, src_v, dst_v, rows_v,
             acc, g_sh, sem):
        cid = lax.axis_index("c")
        sid = lax.axis_index("s")
        wid = sid * NC + cid
        base = wid * EPT
        pltpu.sync_copy(edge_hbm.at[1, pl.ds(base, EPT)],
                        dst_v.at[pl.ds(0, EPT)])
        padn = jnp.full((16,), N, jnp.int32)
        for k in range((EPT_PAD - EPT) // 16):
            dst_v[pl.ds(EPT + k * 16, 16)] = padn
        if with_gather:
            pltpu.sync_copy(edge_hbm.at[0, pl.ds(base, EPT)],
                            src_v.at[pl.ds(0, EPT)])
            zero16 = jnp.zeros((16,), jnp.int32)
            for k in range((EPT_PAD - EPT) // 16):
                src_v[pl.ds(EPT + k * 16, 16)] = zero16
            @pl.when(sid < 10)
            def _():
                pltpu.sync_copy(g_hbm.at[pl.ds(sid * 1000, 1000)],
                                g_sh.at[pl.ds(sid * 1000, 1000)])
        else:
            pltpu.sync_copy(g_hbm, rows_v)
        pltpu.sync_copy(z_hbm.at[pl.ds(sid * ZR, ZR)],
                        acc.at[pl.ds(sid * ZR, ZR)])
        plsc.subcore_barrier()

        if with_gather:
            def body(j, carry):
                pltpu.async_copy(g_sh.at[src_v.at[pl.ds(j * CH, CH)]],
                                 rows_v, sem).wait()
                pltpu.sync_copy(rows_v, acc.at[dst_v.at[pl.ds(j * CH, CH)]],
                                add=True)
                return carry
        else:
            def body(j, carry):
                pltpu.sync_copy(rows_v, acc.at[dst_v.at[pl.ds(j * CH, CH)]],
                                add=True)
                return carry

        lax.fori_loop(0, NCHUNK, body, 0)

        plsc.subcore_barrier()
        pltpu.sync_copy(acc.at[pl.ds(sid * ZR, ZR)],
                        out_hbm.at[pl.ds(cid * N_PAD + sid * ZR, ZR)])

    return scat


_sc_deg = _make_sc_scatter(16, with_gather=False)
_sc_scat64 = _make_sc_scatter(64, with_gather=True)
_sc_scat48 = _make_sc_scatter(48, with_gather=True)


def _full(shape):
    return pl.BlockSpec(shape, lambda i: (0,) * len(shape))


def _rows(d):
    return pl.BlockSpec((BR, d), lambda i: (i, 0))


def _inv_of(p0, p1):
    return lax.rsqrt(p0[:, :1] + p1[:, :1] + 1.0)


def _tc1_body(p0_r, p1_r, x_r, w1_r, b1_r, w2_r, b2_r, wg1_r, o_r):
    h = jnp.maximum(jnp.dot(x_r[...], w1_r[...],
                            preferred_element_type=jnp.float32) + b1_r[...], 0.0)
    h = jnp.maximum(jnp.dot(h, w2_r[...],
                            preferred_element_type=jnp.float32) + b2_r[...], 0.0)
    o_r[...] = jnp.dot(h, wg1_r[...],
                       preferred_element_type=jnp.float32) * _inv_of(
                           p0_r[...], p1_r[...])


def _tc3_body(p0_r, p1_r, s0_r, s1_r, g_r, b1_r, w2_r, o_r):
    inv = _inv_of(p0_r[...], p1_r[...])
    out1 = jnp.maximum((s0_r[...] + s1_r[...] + g_r[...]) * inv + b1_r[...], 0.0)
    o_r[...] = jnp.dot(out1, w2_r[...],
                       preferred_element_type=jnp.float32) * inv


def _tc4_body(p0_r, p1_r, s0_r, s1_r, g_r, b2_r, o_r):
    inv = _inv_of(p0_r[...], p1_r[...])
    z = ((s0_r[...] + s1_r[...] + g_r[...]) * inv)[:, :40] + b2_r[...]
    m = jnp.max(z, axis=1, keepdims=True)
    e = jnp.exp(z - m)
    o_r[...] = (z - m) - jnp.log(jnp.sum(e, axis=1, keepdims=True))


def kernel(x, edge_index, W_fse1, b_fse1, W_fse2, b_fse2, W_gc1, b_gc1,
           W_gc2, b_gc2):
    f32 = jnp.float32
    ei = edge_index.astype(jnp.int32)

    grid = N // BR

    ones16 = jnp.ones((CH, 16), f32)
    z16 = jnp.zeros((N_PAD, 16), f32)
    degp = _sc_deg(ei, ones16, z16)
    p0, p1 = degp[:N], degp[N_PAD:N_PAD + N]

    g1 = pl.pallas_call(
        _tc1_body,
        grid=(grid,),
        in_specs=[_rows(16), _rows(16), _rows(128), _full((128, 16)),
                  _full((1, 16)), _full((16, 16)), _full((1, 16)),
                  _full((16, 64))],
        out_specs=_rows(64),
        out_shape=jax.ShapeDtypeStruct((N, 64), f32),
    )(p0, p1, x, W_fse1, b_fse1.reshape(1, 16), W_fse2,
      b_fse2.reshape(1, 16), W_gc1)

    z64 = jnp.zeros((N_PAD, 64), f32)
    s1 = _sc_scat64(ei, g1, z64)

    w2p = jnp.pad(W_gc2, ((0, 0), (0, 8)))
    g2 = pl.pallas_call(
        _tc3_body,
        grid=(grid,),
        in_specs=[_rows(16), _rows(16), _rows(64), _rows(64), _rows(64),
                  _full((1, 64)), _full((64, 48))],
        out_specs=_rows(48),
        out_shape=jax.ShapeDtypeStruct((N, 48), f32),
    )(p0, p1, s1[:N], s1[N_PAD:N_PAD + N], g1, b_gc1.reshape(1, 64), w2p)

    z48 = jnp.zeros((N_PAD, 48), f32)
    s2 = _sc_scat48(ei, g2, z48)

    out = pl.pallas_call(
        _tc4_body,
        grid=(grid,),
        in_specs=[_rows(16), _rows(16), _rows(48), _rows(48), _rows(48),
                  _full((1, 40))],
        out_specs=_rows(40),
        out_shape=jax.ShapeDtypeStruct((N, 40), f32),
    )(p0, p1, s2[:N], s2[N_PAD:N_PAD + N], g2, b_gc2.reshape(1, 40))

    return out


# speedup vs baseline: 1.4443x; 1.0726x over previous
import functools

import jax
import jax.numpy as jnp
from jax import lax
from jax.experimental import pallas as pl
from jax.experimental.pallas import tpu as pltpu, tpu_sc as plsc

N = 10000
NE = 320000
NC = 2
NS = 16
NW = NC * NS
CH = 128
NCHUNK = 79
EPT = NE // NW
EPT_PAD = NCHUNK * CH
N_PAD = 10240
ZR = N_PAD // NS
BR = 2000


def _make_sc_scatter(D, with_gather):
    mesh = plsc.VectorSubcoreMesh(core_axis_name="c", subcore_axis_name="s")
    scratch = [
        pltpu.VMEM((EPT_PAD,), jnp.int32),
        pltpu.VMEM((EPT_PAD,), jnp.int32),
        pltpu.VMEM((CH, D), jnp.float32),
        pltpu.VMEM_SHARED((N_PAD, D), jnp.float32),
        pltpu.VMEM_SHARED((N, D) if with_gather else (1, D), jnp.float32),
        pltpu.SemaphoreType.DMA,
    ]

    @functools.partial(
        pl.kernel, mesh=mesh,
        out_type=jax.ShapeDtypeStruct((NC * N_PAD, D), jnp.float32),
        scratch_types=scratch,
        compiler_params=pltpu.CompilerParams(use_tc_tiling_on_sc=False),
    )
    def scat(edge_hbm, g_hbm, z_hbm, out_hbm, src_v, dst_v, rows_v,
             acc, g_sh, sem):
        cid = lax.axis_index("c")
        sid = lax.axis_index("s")
        wid = sid * NC + cid
        base = wid * EPT
        pltpu.sync_copy(edge_hbm.at[1, pl.ds(base, EPT)],
                        dst_v.at[pl.ds(0, EPT)])
        padn = jnp.full((16,), N, jnp.int32)
        for k in range((EPT_PAD - EPT) // 16):
            dst_v[pl.ds(EPT + k * 16, 16)] = padn
        if with_gather:
            pltpu.sync_copy(edge_hbm.at[0, pl.ds(base, EPT)],
                            src_v.at[pl.ds(0, EPT)])
            zero16 = jnp.zeros((16,), jnp.int32)
            for k in range((EPT_PAD - EPT) // 16):
                src_v[pl.ds(EPT + k * 16, 16)] = zero16
            @pl.when(sid < 10)
            def _():
                pltpu.sync_copy(g_hbm.at[pl.ds(sid * 1000, 1000)],
                                g_sh.at[pl.ds(sid * 1000, 1000)])
        else:
            pltpu.sync_copy(g_hbm, rows_v)
        pltpu.sync_copy(z_hbm.at[pl.ds(sid * ZR, ZR)],
                        acc.at[pl.ds(sid * ZR, ZR)])
        plsc.subcore_barrier()

        if with_gather:
            def body(j, carry):
                pltpu.async_copy(g_sh.at[src_v.at[pl.ds(j * CH, CH)]],
                                 rows_v, sem).wait()
                pltpu.sync_copy(rows_v, acc.at[dst_v.at[pl.ds(j * CH, CH)]],
                                add=True)
                return carry
        else:
            def body(j, carry):
                pltpu.sync_copy(rows_v, acc.at[dst_v.at[pl.ds(j * CH, CH)]],
                                add=True)
                return carry

        lax.fori_loop(0, NCHUNK, body, 0)

        plsc.subcore_barrier()
        pltpu.sync_copy(acc.at[pl.ds(sid * ZR, ZR)],
                        out_hbm.at[pl.ds(cid * N_PAD + sid * ZR, ZR)])

    return scat


_sc_deg = _make_sc_scatter(16, with_gather=False)
_sc_scat64 = _make_sc_scatter(64, with_gather=True)
_sc_scat48 = _make_sc_scatter(48, with_gather=True)


def _full(shape):
    return pl.BlockSpec(shape, lambda i: (0,) * len(shape))


def _rows(d):
    return pl.BlockSpec((BR, d), lambda i: (i, 0))


def _part(d, c):
    return pl.BlockSpec((1, BR, d), lambda i, c=c: (c, i, 0))


def _inv_of(p0, p1):
    return lax.rsqrt(p0[0][:, :1] + p1[0][:, :1] + 1.0)


def _tc1_body(p0_r, p1_r, x_r, w1_r, b1_r, w2_r, b2_r, wg1_r, o_r):
    h = jnp.maximum(jnp.dot(x_r[...], w1_r[...],
                            preferred_element_type=jnp.float32) + b1_r[...], 0.0)
    h = jnp.maximum(jnp.dot(h, w2_r[...],
                            preferred_element_type=jnp.float32) + b2_r[...], 0.0)
    o_r[...] = jnp.dot(h, wg1_r[...],
                       preferred_element_type=jnp.float32) * _inv_of(
                           p0_r, p1_r)


def _tc3_body(p0_r, p1_r, s0_r, s1_r, g_r, b1_r, w2_r, o_r):
    inv = _inv_of(p0_r, p1_r)
    out1 = jnp.maximum((s0_r[0] + s1_r[0] + g_r[...]) * inv + b1_r[...], 0.0)
    o_r[...] = jnp.dot(out1, w2_r[...],
                       preferred_element_type=jnp.float32) * inv


def _tc4_body(p0_r, p1_r, s0_r, s1_r, g_r, b2_r, o_r):
    inv = _inv_of(p0_r, p1_r)
    z = ((s0_r[0] + s1_r[0] + g_r[...]) * inv)[:, :40] + b2_r[...]
    m = jnp.max(z, axis=1, keepdims=True)
    e = jnp.exp(z - m)
    o_r[...] = (z - m) - jnp.log(jnp.sum(e, axis=1, keepdims=True))


def kernel(x, edge_index, W_fse1, b_fse1, W_fse2, b_fse2, W_gc1, b_gc1,
           W_gc2, b_gc2):
    f32 = jnp.float32
    ei = edge_index.astype(jnp.int32)

    grid = N // BR

    ones16 = jnp.ones((CH, 16), f32)
    z16 = jnp.zeros((N_PAD, 16), f32)
    degp = _sc_deg(ei, ones16, z16).reshape(NC, N_PAD, 16)

    g1 = pl.pallas_call(
        _tc1_body,
        grid=(grid,),
        in_specs=[_part(16, 0), _part(16, 1), _rows(128), _full((128, 16)),
                  _full((1, 16)), _full((16, 16)), _full((1, 16)),
                  _full((16, 64))],
        out_specs=_rows(64),
        out_shape=jax.ShapeDtypeStruct((N, 64), f32),
    )(degp, degp, x, W_fse1, b_fse1.reshape(1, 16), W_fse2,
      b_fse2.reshape(1, 16), W_gc1)

    z64 = jnp.zeros((N_PAD, 64), f32)
    s1 = _sc_scat64(ei, g1, z64).reshape(NC, N_PAD, 64)

    w2p = jnp.pad(W_gc2, ((0, 0), (0, 8)))
    g2 = pl.pallas_call(
        _tc3_body,
        grid=(grid,),
        in_specs=[_part(16, 0), _part(16, 1), _part(64, 0), _part(64, 1),
                  _rows(64), _full((1, 64)), _full((64, 48))],
        out_specs=_rows(48),
        out_shape=jax.ShapeDtypeStruct((N, 48), f32),
    )(degp, degp, s1, s1, g1, b_gc1.reshape(1, 64), w2p)

    z48 = jnp.zeros((N_PAD, 48), f32)
    s2 = _sc_scat48(ei, g2, z48).reshape(NC, N_PAD, 48)

    out = pl.pallas_call(
        _tc4_body,
        grid=(grid,),
        in_specs=[_part(16, 0), _part(16, 1), _part(48, 0), _part(48, 1),
                  _rows(48), _full((1, 40))],
        out_specs=_rows(40),
        out_shape=jax.ShapeDtypeStruct((N, 40), f32),
    )(degp, degp, s2, s2, g2, b_gc2.reshape(1, 40))

    return out


# speedup vs baseline: 1.4470x; 1.0019x over previous
import functools

import jax
import jax.numpy as jnp
from jax import lax
from jax.experimental import pallas as pl
from jax.experimental.pallas import tpu as pltpu, tpu_sc as plsc

N = 10000
NE = 320000
NC = 2
NS = 16
NW = NC * NS
CH = 128
NCHUNK = 79
EPT = NE // NW
EPT_PAD = NCHUNK * CH
N_PAD = 10240
ZR = N_PAD // NS
BR = 2000


def _make_sc_scatter(D, with_gather):
    mesh = plsc.VectorSubcoreMesh(core_axis_name="c", subcore_axis_name="s")
    scratch = [
        pltpu.VMEM((EPT_PAD,), jnp.int32),
        pltpu.VMEM((EPT_PAD,), jnp.int32),
        pltpu.VMEM((CH, D), jnp.float32),
        pltpu.VMEM_SHARED((N_PAD, D), jnp.float32),
        pltpu.VMEM_SHARED((N, D) if with_gather else (1, D), jnp.float32),
        pltpu.SemaphoreType.DMA,
    ]

    @functools.partial(
        pl.kernel, mesh=mesh,
        out_type=jax.ShapeDtypeStruct((NC * N_PAD, D), jnp.float32),
        scratch_types=scratch,
        compiler_params=pltpu.CompilerParams(use_tc_tiling_on_sc=False),
    )
    def scat(edge_hbm, g_hbm, z_hbm, out_hbm, src_v, dst_v, rows_v,
             acc, g_sh, sem):
        cid = lax.axis_index("c")
        sid = lax.axis_index("s")
        wid = sid * NC + cid
        base = wid * EPT
        pltpu.sync_copy(edge_hbm.at[1, pl.ds(base, EPT)],
                        dst_v.at[pl.ds(0, EPT)])
        padn = jnp.full((16,), N, jnp.int32)
        for k in range((EPT_PAD - EPT) // 16):
            dst_v[pl.ds(EPT + k * 16, 16)] = padn
        if with_gather:
            pltpu.sync_copy(edge_hbm.at[0, pl.ds(base, EPT)],
                            src_v.at[pl.ds(0, EPT)])
            zero16 = jnp.zeros((16,), jnp.int32)
            for k in range((EPT_PAD - EPT) // 16):
                src_v[pl.ds(EPT + k * 16, 16)] = zero16
            @pl.when(sid < 10)
            def _():
                pltpu.sync_copy(g_hbm.at[pl.ds(sid * 1000, 1000)],
                                g_sh.at[pl.ds(sid * 1000, 1000)])
        else:
            pltpu.sync_copy(g_hbm, rows_v)
        pltpu.sync_copy(z_hbm.at[pl.ds(sid * ZR, ZR)],
                        acc.at[pl.ds(sid * ZR, ZR)])
        plsc.subcore_barrier()

        if with_gather:
            def body(j, carry):
                pltpu.async_copy(g_sh.at[src_v.at[pl.ds(j * CH, CH)]],
                                 rows_v, sem).wait()
                pltpu.sync_copy(rows_v, acc.at[dst_v.at[pl.ds(j * CH, CH)]],
                                add=True)
                return carry
        else:
            def body(j, carry):
                pltpu.sync_copy(rows_v, acc.at[dst_v.at[pl.ds(j * CH, CH)]],
                                add=True)
                return carry

        lax.fori_loop(0, NCHUNK, body, 0)

        plsc.subcore_barrier()
        pltpu.sync_copy(acc.at[pl.ds(sid * ZR, ZR)],
                        out_hbm.at[pl.ds(cid * N_PAD + sid * ZR, ZR)])

    return scat


_sc_deg = _make_sc_scatter(16, with_gather=False)
_sc_scat64 = _make_sc_scatter(64, with_gather=True)
_sc_scat48 = _make_sc_scatter(48, with_gather=True)


def _full(shape):
    return pl.BlockSpec(shape, lambda i: (0,) * len(shape))


def _rows(d):
    return pl.BlockSpec((BR, d), lambda i: (i, 0))


def _part(d, c):
    return pl.BlockSpec((1, BR, d), lambda i, c=c: (c, i, 0))


def _inv_of(p0, p1):
    return lax.rsqrt(p0[0][:, :1] + p1[0][:, :1] + 1.0)


def _tc1_body(x_r, w1_r, b1_r, w2_r, b2_r, wg1_r, o_r):
    h = jnp.maximum(jnp.dot(x_r[...], w1_r[...],
                            preferred_element_type=jnp.float32) + b1_r[...], 0.0)
    h = jnp.maximum(jnp.dot(h, w2_r[...],
                            preferred_element_type=jnp.float32) + b2_r[...], 0.0)
    o_r[...] = jnp.dot(h, wg1_r[...], preferred_element_type=jnp.float32)


def _tc2_body(p0_r, p1_r, g_r, o_r):
    o_r[...] = g_r[...] * _inv_of(p0_r, p1_r)


def _tc3_body(p0_r, p1_r, s0_r, s1_r, g_r, b1_r, w2_r, o_r):
    inv = _inv_of(p0_r, p1_r)
    out1 = jnp.maximum((s0_r[0] + s1_r[0] + g_r[...]) * inv + b1_r[...], 0.0)
    o_r[...] = jnp.dot(out1, w2_r[...],
                       preferred_element_type=jnp.float32) * inv


def _tc4_body(p0_r, p1_r, s0_r, s1_r, g_r, b2_r, o_r):
    inv = _inv_of(p0_r, p1_r)
    z = ((s0_r[0] + s1_r[0] + g_r[...]) * inv)[:, :40] + b2_r[...]
    m = jnp.max(z, axis=1, keepdims=True)
    e = jnp.exp(z - m)
    o_r[...] = (z - m) - jnp.log(jnp.sum(e, axis=1, keepdims=True))


def kernel(x, edge_index, W_fse1, b_fse1, W_fse2, b_fse2, W_gc1, b_gc1,
           W_gc2, b_gc2):
    f32 = jnp.float32
    ei = edge_index.astype(jnp.int32)

    grid = N // BR

    ones16 = jnp.ones((CH, 16), f32)
    z16 = jnp.zeros((N_PAD, 16), f32)
    degp = _sc_deg(ei, ones16, z16).reshape(NC, N_PAD, 16)

    g1_raw = pl.pallas_call(
        _tc1_body,
        grid=(grid,),
        in_specs=[_rows(128), _full((128, 16)), _full((1, 16)),
                  _full((16, 16)), _full((1, 16)), _full((16, 64))],
        out_specs=_rows(64),
        out_shape=jax.ShapeDtypeStruct((N, 64), f32),
    )(x, W_fse1, b_fse1.reshape(1, 16), W_fse2, b_fse2.reshape(1, 16),
      W_gc1)

    g1 = pl.pallas_call(
        _tc2_body,
        grid=(grid,),
        in_specs=[_part(16, 0), _part(16, 1), _rows(64)],
        out_specs=_rows(64),
        out_shape=jax.ShapeDtypeStruct((N, 64), f32),
    )(degp, degp, g1_raw)

    z64 = jnp.zeros((N_PAD, 64), f32)
    s1 = _sc_scat64(ei, g1, z64).reshape(NC, N_PAD, 64)

    w2p = jnp.pad(W_gc2, ((0, 0), (0, 8)))
    g2 = pl.pallas_call(
        _tc3_body,
        grid=(grid,),
        in_specs=[_part(16, 0), _part(16, 1), _part(64, 0), _part(64, 1),
                  _rows(64), _full((1, 64)), _full((64, 48))],
        out_specs=_rows(48),
        out_shape=jax.ShapeDtypeStruct((N, 48), f32),
    )(degp, degp, s1, s1, g1, b_gc1.reshape(1, 64), w2p)

    z48 = jnp.zeros((N_PAD, 48), f32)
    s2 = _sc_scat48(ei, g2, z48).reshape(NC, N_PAD, 48)

    out = pl.pallas_call(
        _tc4_body,
        grid=(grid,),
        in_specs=[_part(16, 0), _part(16, 1), _part(48, 0), _part(48, 1),
                  _rows(48), _full((1, 40))],
        out_specs=_rows(40),
        out_shape=jax.ShapeDtypeStruct((N, 40), f32),
    )(degp, degp, s2, s2, g2, b_gc2.reshape(1, 40))

    return out
